# speedup vs baseline: 3.3305x; 3.3305x over previous
import functools

import jax
import jax.numpy as jnp
from jax import lax
from jax.experimental import pallas as pl
from jax.experimental.pallas import tpu as pltpu
from jax.experimental.pallas import tpu_sc as plsc

_C = 128
_NBUF = 2


@functools.lru_cache(maxsize=None)
def _gather_kernel(B, D, NW, NCH):
    mesh = plsc.VectorSubcoreMesh(core_axis_name="c", subcore_axis_name="s")
    rows_per_w = NCH * _C

    @functools.partial(
        pl.kernel,
        out_type=jax.ShapeDtypeStruct((B, D), jnp.float32),
        mesh=mesh,
        scratch_types=[
            pltpu.VMEM((NCH, _C), jnp.int32),
            *[pltpu.VMEM((_C, D), jnp.float32) for _ in range(_NBUF)],
            *[pltpu.SemaphoreType.DMA for _ in range(2 * _NBUF)],
        ],
    )
    def k(table_hbm, idx_hbm, out_hbm, idx_v, *rest):
        bufs = rest[:_NBUF]
        gsem = rest[_NBUF:2 * _NBUF]
        wsem = rest[2 * _NBUF:3 * _NBUF]
        wid = lax.axis_index("s") * 2 + lax.axis_index("c")
        base = wid * rows_per_w

        pltpu.sync_copy(idx_hbm.at[wid], idx_v)

        for s in range(_NBUF):
            pltpu.async_copy(table_hbm.at[idx_v.at[s]], bufs[s], gsem[s])

        @pl.loop(0, NCH // _NBUF)
        def _(i):
            for s in range(_NBUF):
                ch = i * _NBUF + s
                pltpu.make_async_copy(
                    table_hbm.at[pl.ds(0, _C)], bufs[s], gsem[s]).wait()
                pltpu.async_copy(
                    bufs[s], out_hbm.at[pl.ds(base + ch * _C, _C)], wsem[s])
                nxt = ch + _NBUF

                @pl.when(nxt < NCH)
                def _():
                    pltpu.make_async_copy(
                        bufs[s], out_hbm.at[pl.ds(base, _C)], wsem[s]).wait()
                    pltpu.async_copy(
                        table_hbm.at[idx_v.at[nxt]], bufs[s], gsem[s])

        for s in range(_NBUF):
            pltpu.make_async_copy(
                bufs[s], out_hbm.at[pl.ds(base, _C)], wsem[s]).wait()

    return k


def kernel(input, table):
    B, L = input.shape
    V, D = table.shape
    total = B * L
    NW = 32
    rows_per_w = total // NW
    NCH = rows_per_w // _C
    idx = input.reshape(NW, NCH, _C).astype(jnp.int32)
    out = _gather_kernel(total, D, NW, NCH)(table, idx)
    return out.reshape(B, L, D)


# speedup vs baseline: 3.3439x; 1.0040x over previous
import functools

import jax
import jax.numpy as jnp
from jax import lax
from jax.experimental import pallas as pl
from jax.experimental.pallas import tpu as pltpu
from jax.experimental.pallas import tpu_sc as plsc

_C = 128
_NBUF = 5


@functools.lru_cache(maxsize=None)
def _gather_kernel(B, D, NW, NCH):
    mesh = plsc.VectorSubcoreMesh(core_axis_name="c", subcore_axis_name="s")
    rows_per_w = NCH * _C

    @functools.partial(
        pl.kernel,
        out_type=jax.ShapeDtypeStruct((B, D), jnp.float32),
        mesh=mesh,
        scratch_types=[
            pltpu.VMEM((NCH, _C), jnp.int32),
            *[pltpu.VMEM((_C, D), jnp.float32) for _ in range(_NBUF)],
            *[pltpu.SemaphoreType.DMA for _ in range(2 * _NBUF)],
        ],
    )
    def k(table_hbm, idx_hbm, out_hbm, idx_v, *rest):
        bufs = rest[:_NBUF]
        gsem = rest[_NBUF:2 * _NBUF]
        wsem = rest[2 * _NBUF:3 * _NBUF]
        wid = lax.axis_index("s") * 2 + lax.axis_index("c")
        base = wid * rows_per_w

        pltpu.sync_copy(idx_hbm.at[wid], idx_v)

        for s in range(_NBUF):
            pltpu.async_copy(table_hbm.at[idx_v.at[s]], bufs[s], gsem[s])

        @pl.loop(0, NCH // _NBUF)
        def _(i):
            for s in range(_NBUF):
                ch = i * _NBUF + s
                pltpu.make_async_copy(
                    table_hbm.at[pl.ds(0, _C)], bufs[s], gsem[s]).wait()
                pltpu.async_copy(
                    bufs[s], out_hbm.at[pl.ds(base + ch * _C, _C)], wsem[s])
                nxt = ch + _NBUF

                @pl.when(nxt < NCH)
                def _():
                    pltpu.make_async_copy(
                        bufs[s], out_hbm.at[pl.ds(base, _C)], wsem[s]).wait()
                    pltpu.async_copy(
                        table_hbm.at[idx_v.at[nxt]], bufs[s], gsem[s])

        for s in range(_NBUF):
            pltpu.make_async_copy(
                bufs[s], out_hbm.at[pl.ds(base, _C)], wsem[s]).wait()

    return k


def kernel(input, table):
    B, L = input.shape
    V, D = table.shape
    total = B * L
    NW = 32
    rows_per_w = total // NW
    NCH = rows_per_w // _C
    idx = input.reshape(NW, NCH, _C).astype(jnp.int32)
    out = _gather_kernel(total, D, NW, NCH)(table, idx)
    return out.reshape(B, L, D)


# speedup vs baseline: 3.3543x; 1.0031x over previous
import functools

import jax
import jax.numpy as jnp
from jax import lax
from jax.experimental import pallas as pl
from jax.experimental.pallas import tpu as pltpu
from jax.experimental.pallas import tpu_sc as plsc

_C = 128
_NBUF = 5


@functools.lru_cache(maxsize=None)
def _gather_kernel(B, D, NW, NCH):
    mesh = plsc.VectorSubcoreMesh(core_axis_name="c", subcore_axis_name="s")
    rows_per_w = NCH * _C

    @functools.partial(
        pl.kernel,
        out_type=jax.ShapeDtypeStruct((B, D), jnp.float32),
        mesh=mesh,
        scratch_types=[
            pltpu.VMEM((NCH * _C,), jnp.int32),
            *[pltpu.VMEM((_C, D), jnp.float32) for _ in range(_NBUF)],
            *[pltpu.SemaphoreType.DMA for _ in range(2 * _NBUF)],
        ],
        compiler_params=pltpu.CompilerParams(use_tc_tiling_on_sc=True),
    )
    def k(table_hbm, idx_hbm, out_hbm, idx_v, *rest):
        bufs = rest[:_NBUF]
        gsem = rest[_NBUF:2 * _NBUF]
        wsem = rest[2 * _NBUF:3 * _NBUF]
        wid = lax.axis_index("s") * 2 + lax.axis_index("c")
        base = wid * rows_per_w

        pltpu.sync_copy(idx_hbm.at[pl.ds(base, rows_per_w)], idx_v)

        for s in range(_NBUF):
            pltpu.async_copy(
                table_hbm.at[idx_v.at[pl.ds(s * _C, _C)]], bufs[s], gsem[s])

        @pl.loop(0, NCH // _NBUF)
        def _(i):
            for s in range(_NBUF):
                ch = i * _NBUF + s
                pltpu.make_async_copy(
                    table_hbm.at[pl.ds(0, _C)], bufs[s], gsem[s]).wait()
                pltpu.async_copy(
                    bufs[s], out_hbm.at[pl.ds(base + ch * _C, _C)], wsem[s])
                nxt = ch + _NBUF

                @pl.when(nxt < NCH)
                def _():
                    pltpu.make_async_copy(
                        bufs[s], out_hbm.at[pl.ds(base, _C)], wsem[s]).wait()
                    pltpu.async_copy(
                        table_hbm.at[idx_v.at[pl.ds(nxt * _C, _C)]],
                        bufs[s], gsem[s])

        for s in range(_NBUF):
            pltpu.make_async_copy(
                bufs[s], out_hbm.at[pl.ds(base, _C)], wsem[s]).wait()

    return k


def kernel(input, table):
    B, L = input.shape
    V, D = table.shape
    total = B * L
    NW = 32
    rows_per_w = total // NW
    NCH = rows_per_w // _C
    idx = input.reshape(total).astype(jnp.int32)
    out = _gather_kernel(total, D, NW, NCH)(table, idx)
    return out.reshape(B, L, D)


# speedup vs baseline: 10.7918x; 3.2173x over previous
import functools

import jax
import jax.numpy as jnp
from jax import lax
from jax.experimental import pallas as pl
from jax.experimental.pallas import tpu as pltpu
from jax.experimental.pallas import tpu_sc as plsc

_C = 128
_NBUF = 5


@functools.lru_cache(maxsize=None)
def _gather_kernel(B, L, V, D, NW):
    mesh = plsc.VectorSubcoreMesh(core_axis_name="c", subcore_axis_name="s")

    @functools.partial(
        pl.kernel,
        out_type=jax.ShapeDtypeStruct((B * L, D), jnp.float32),
        mesh=mesh,
        scratch_types=[
            pltpu.VMEM((L, _C), jnp.int32),
            *[pltpu.VMEM((_C, D), jnp.float32) for _ in range(_NBUF)],
            *[pltpu.SemaphoreType.DMA for _ in range(2 * _NBUF)],
        ],
        compiler_params=pltpu.CompilerParams(use_tc_tiling_on_sc=True),
    )
    def k(table_hbm, idx_hbm, out_hbm, idx_v, *rest):
        bufs = rest[:_NBUF]
        gsem = rest[_NBUF:2 * _NBUF]
        wsem = rest[2 * _NBUF:3 * _NBUF]
        wid = lax.axis_index("s") * 2 + lax.axis_index("c")
        col = wid * _C

        pltpu.sync_copy(idx_hbm.at[:, pl.ds(col, _C)], idx_v)

        for s in range(_NBUF):
            pltpu.async_copy(table_hbm.at[idx_v.at[s]], bufs[s], gsem[s])

        @pl.loop(0, L // _NBUF)
        def _(i):
            for s in range(_NBUF):
                ch = i * _NBUF + s
                pltpu.make_async_copy(
                    table_hbm.at[pl.ds(0, _C)], bufs[s], gsem[s]).wait()
                pltpu.async_copy(
                    bufs[s], out_hbm.at[pl.ds(ch * B + col, _C)], wsem[s])
                nxt = ch + _NBUF

                @pl.when(nxt < L)
                def _():
                    pltpu.make_async_copy(
                        bufs[s], out_hbm.at[pl.ds(col, _C)], wsem[s]).wait()
                    pltpu.async_copy(
                        table_hbm.at[idx_v.at[nxt]], bufs[s], gsem[s])

        for s in range(_NBUF):
            pltpu.make_async_copy(
                bufs[s], out_hbm.at[pl.ds(col, _C)], wsem[s]).wait()

    return k


def kernel(input, table):
    B, L = input.shape
    V, D = table.shape
    NW = 32
    idx = input.T.astype(jnp.int32)
    out = _gather_kernel(B, L, V, D, NW)(table, idx)
    return out.reshape(L, B, D).transpose(1, 0, 2)
